# baseline (speedup 1.0000x reference)
import functools

import jax
import jax.numpy as jnp
from jax import lax
from jax.experimental import pallas as pl
from jax.experimental.pallas import tpu as pltpu
from jax.experimental.pallas import tpu_sc as plsc

F32 = jnp.float32
KNB = 20
KPAD = 32
NEG = -3.0e38
EPS = 1e-5


def _gather_rows(table, idx, ch):
    V, D = table.shape
    (T,) = idx.shape
    info = plsc.get_sparse_core_info()
    nc_, ns_ = info.num_cores, info.num_subcores
    nw = nc_ * ns_
    per_w = T // nw
    assert T % nw == 0 and per_w % ch == 0 and ch % 8 == 0
    nch = per_w // ch
    mesh = plsc.VectorSubcoreMesh(core_axis_name="c", subcore_axis_name="s")

    @functools.partial(
        pl.kernel,
        mesh=mesh,
        out_type=jax.ShapeDtypeStruct((T, D), F32),
        scratch_types=[
            pltpu.VMEM((ch,), jnp.int32),
            pltpu.VMEM((ch, D), F32),
            pltpu.SemaphoreType.DMA,
        ],
    )
    def k(table_hbm, idx_hbm, out_hbm, idx_v, rows_v, sem):
        wid = lax.axis_index("s") * nc_ + lax.axis_index("c")
        base = wid * per_w
        for c in range(nch):
            off = base + c * ch
            pltpu.sync_copy(idx_hbm.at[pl.ds(off, ch)], idx_v)
            pltpu.async_copy(table_hbm.at[idx_v], rows_v, sem).wait()
            pltpu.sync_copy(rows_v, out_hbm.at[pl.ds(off, ch)])

    return k(table, idx)


def _knn_call(q, dbt, mb):
    B, M, C = q.shape
    N = dbt.shape[2]

    def body(q_ref, dbt_ref, o_ref):
        qb = q_ref[0]
        db = dbt_ref[0]
        pd = 2.0 * jnp.dot(qb, db, preferred_element_type=F32)
        pd = pd - jnp.sum(qb * qb, axis=1, keepdims=True)
        pd = pd - jnp.sum(db * db, axis=0, keepdims=True)
        ii = lax.broadcasted_iota(jnp.int32, (mb, N), 1)
        col = lax.broadcasted_iota(jnp.int32, (mb, KPAD), 1)
        acc = jnp.zeros((mb, KPAD), jnp.int32)
        for j in range(KNB):
            m = jnp.max(pd, axis=1, keepdims=True)
            am = jnp.min(jnp.where(pd == m, ii, N), axis=1, keepdims=True)
            acc = jnp.where(col == j, am, acc)
            pd = jnp.where(ii == am, NEG, pd)
        o_ref[0] = acc

    return pl.pallas_call(
        body,
        grid=(B, M // mb),
        in_specs=[
            pl.BlockSpec((1, mb, C), lambda b, m: (b, m, 0)),
            pl.BlockSpec((1, C, N), lambda b, m: (b, 0, 0)),
        ],
        out_specs=pl.BlockSpec((1, mb, KPAD), lambda b, m: (b, m, 0)),
        out_shape=jax.ShapeDtypeStruct((B, M, KPAD), jnp.int32),
    )(q, dbt)


def _lrelu(y):
    return jnp.where(y >= 0, y, 0.2 * y)


def _group_bcast(s, gm, gmt):
    return jnp.dot(jnp.dot(s, gm, preferred_element_type=F32,
                           precision=lax.Precision.HIGHEST), gmt,
                   preferred_element_type=F32, precision=lax.Precision.HIGHEST)


def _norm_act(y, s0, s1, gm, gmt, g, b, inv_cnt):
    mean = _group_bcast(s0, gm, gmt) * inv_cnt
    msq = _group_bcast(s1, gm, gmt) * inv_cnt
    var = msq - mean * mean
    return _lrelu((y - mean) / jnp.sqrt(var + EPS) * g + b)


def _conv_stats_call(nbr, ctr, ctrsh, wt, nc):
    B, S, C = nbr.shape
    M = ctr.shape[1]
    O = wt.shape[1]
    R = S // nc
    T = R // M

    def body(n_ref, c_ref, cs_ref, w_ref, y_ref, st_ref):
        ci = pl.program_id(1)
        ct = c_ref[0]
        csh = cs_ref[0]
        s0 = jnp.zeros((1, O), F32)
        s1 = jnp.zeros((1, O), F32)
        for t in range(T):
            x = n_ref[0, pl.ds(t * M, M), :]
            y = jnp.dot(x - ct + csh, w_ref[...], preferred_element_type=F32)
            y_ref[0, pl.ds(t * M, M), :] = y
            s0 = s0 + jnp.sum(y, axis=0, keepdims=True)
            s1 = s1 + jnp.sum(y * y, axis=0, keepdims=True)

        @pl.when(ci == 0)
        def _():
            st_ref[0, 0:1, :] = s0
            st_ref[0, 1:2, :] = s1

        @pl.when(ci != 0)
        def _():
            st_ref[0, 0:1, :] = st_ref[0, 0:1, :] + s0
            st_ref[0, 1:2, :] = st_ref[0, 1:2, :] + s1

    return pl.pallas_call(
        body,
        grid=(B, nc),
        in_specs=[
            pl.BlockSpec((1, R, C), lambda b, c: (b, c, 0)),
            pl.BlockSpec((1, M, C), lambda b, c: (b, 0, 0)),
            pl.BlockSpec((1, M, C), lambda b, c: (b, 0, 0)),
            pl.BlockSpec((C, O), lambda b, c: (0, 0)),
        ],
        out_specs=[
            pl.BlockSpec((1, R, O), lambda b, c: (b, c, 0)),
            pl.BlockSpec((1, 2, O), lambda b, c: (b, 0, 0)),
        ],
        out_shape=[
            jax.ShapeDtypeStruct((B, S, O), F32),
            jax.ShapeDtypeStruct((B, 2, O), F32),
        ],
    )(nbr, ctr, ctrsh, wt)


def _norm_conv_stats_call(y, st, g, b, gm, gmt, w2t, cnt, nc):
    B, S, O = y.shape
    O2 = w2t.shape[1]
    R = S // nc
    inv_cnt = 1.0 / cnt

    def body(y_ref, st_ref, g_ref, b_ref, gm_ref, gmt_ref, w_ref, y2_ref, st2_ref):
        ci = pl.program_id(1)
        a = _norm_act(y_ref[0], st_ref[0, 0:1, :], st_ref[0, 1:2, :],
                      gm_ref[...], gmt_ref[...], g_ref[...], b_ref[...], inv_cnt)
        y2 = jnp.dot(a, w_ref[...], preferred_element_type=F32)
        y2_ref[0] = y2
        s0 = jnp.sum(y2, axis=0, keepdims=True)
        s1 = jnp.sum(y2 * y2, axis=0, keepdims=True)

        @pl.when(ci == 0)
        def _():
            st2_ref[0, 0:1, :] = s0
            st2_ref[0, 1:2, :] = s1

        @pl.when(ci != 0)
        def _():
            st2_ref[0, 0:1, :] = st2_ref[0, 0:1, :] + s0
            st2_ref[0, 1:2, :] = st2_ref[0, 1:2, :] + s1

    return pl.pallas_call(
        body,
        grid=(B, nc),
        in_specs=[
            pl.BlockSpec((1, R, O), lambda b, c: (b, c, 0)),
            pl.BlockSpec((1, 2, O), lambda b, c: (b, 0, 0)),
            pl.BlockSpec((1, O), lambda b, c: (0, 0)),
            pl.BlockSpec((1, O), lambda b, c: (0, 0)),
            pl.BlockSpec((O, 32), lambda b, c: (0, 0)),
            pl.BlockSpec((32, O), lambda b, c: (0, 0)),
            pl.BlockSpec((O, O2), lambda b, c: (0, 0)),
        ],
        out_specs=[
            pl.BlockSpec((1, R, O2), lambda b, c: (b, c, 0)),
            pl.BlockSpec((1, 2, O2), lambda b, c: (b, 0, 0)),
        ],
        out_shape=[
            jax.ShapeDtypeStruct((B, S, O2), F32),
            jax.ShapeDtypeStruct((B, 2, O2), F32),
        ],
    )(y, st, g, b, gm, gmt, w2t)


def _norm_maxk_call(y, st, g, b, gm, gmt, cnt, M, nc):
    B, S, O = y.shape
    R = S // nc
    T = R // M
    inv_cnt = 1.0 / cnt

    def body(y_ref, st_ref, g_ref, b_ref, gm_ref, gmt_ref, o_ref):
        ci = pl.program_id(1)
        a = _norm_act(y_ref[0], st_ref[0, 0:1, :], st_ref[0, 1:2, :],
                      gm_ref[...], gmt_ref[...], g_ref[...], b_ref[...], inv_cnt)
        m = lax.slice(a, (0, 0), (M, O))
        for t in range(1, T):
            m = jnp.maximum(m, lax.slice(a, (t * M, 0), ((t + 1) * M, O)))

        @pl.when(ci == 0)
        def _():
            o_ref[0] = m

        @pl.when(ci != 0)
        def _():
            o_ref[0] = jnp.maximum(o_ref[0], m)

    return pl.pallas_call(
        body,
        grid=(B, nc),
        in_specs=[
            pl.BlockSpec((1, R, O), lambda b, c: (b, c, 0)),
            pl.BlockSpec((1, 2, O), lambda b, c: (b, 0, 0)),
            pl.BlockSpec((1, O), lambda b, c: (0, 0)),
            pl.BlockSpec((1, O), lambda b, c: (0, 0)),
            pl.BlockSpec((O, 32), lambda b, c: (0, 0)),
            pl.BlockSpec((32, O), lambda b, c: (0, 0)),
        ],
        out_specs=pl.BlockSpec((1, M, O), lambda b, c: (b, 0, 0)),
        out_shape=jax.ShapeDtypeStruct((B, M, O), F32),
    )(y, st, g, b, gm, gmt)


def _tail_call(x, w6t, g6, b6, gm6, gm6t, w7at, w7bt, g7, b7, gm7, gm7t,
               w8t, g8, b8, gm8, gm8t):
    B, NK, _ = x.shape

    def body(x_ref, w6_ref, g6_ref, b6_ref, gm6_ref, gm6t_ref,
             w7a_ref, w7b_ref, g7_ref, b7_ref, gm7_ref, gm7t_ref,
             w8_ref, g8_ref, b8_ref, gm8_ref, gm8t_ref, o_ref):
        xb = x_ref[0]
        y6 = jnp.dot(xb, w6_ref[...], preferred_element_type=F32)
        a6 = _norm_act(y6, jnp.sum(y6, axis=0, keepdims=True),
                       jnp.sum(y6 * y6, axis=0, keepdims=True),
                       gm6_ref[...], gm6t_ref[...], g6_ref[...], b6_ref[...],
                       1.0 / (32.0 * NK))
        gmax = jnp.max(a6, axis=0, keepdims=True)
        y7 = (jnp.dot(xb, w7b_ref[...], preferred_element_type=F32)
              + jnp.dot(gmax, w7a_ref[...], preferred_element_type=F32))
        a7 = _norm_act(y7, jnp.sum(y7, axis=0, keepdims=True),
                       jnp.sum(y7 * y7, axis=0, keepdims=True),
                       gm7_ref[...], gm7t_ref[...], g7_ref[...], b7_ref[...],
                       1.0 / (16.0 * NK))
        y8 = jnp.dot(a7, w8_ref[...], preferred_element_type=F32)
        a8 = _norm_act(y8, jnp.sum(y8, axis=0, keepdims=True),
                       jnp.sum(y8 * y8, axis=0, keepdims=True),
                       gm8_ref[...], gm8t_ref[...], g8_ref[...], b8_ref[...],
                       1.0 / (4.0 * NK))
        o_ref[0] = a8

    full = lambda shape: pl.BlockSpec(shape, lambda b: tuple(0 for _ in shape))
    return pl.pallas_call(
        body,
        grid=(B,),
        in_specs=[
            pl.BlockSpec((1, NK, 192), lambda b: (b, 0, 0)),
            full((192, 1024)), full((1, 1024)), full((1, 1024)),
            full((1024, 32)), full((32, 1024)),
            full((1024, 512)), full((192, 512)), full((1, 512)), full((1, 512)),
            full((512, 32)), full((32, 512)),
            full((512, 128)), full((1, 128)), full((1, 128)),
            full((128, 32)), full((32, 128)),
        ],
        out_specs=pl.BlockSpec((1, NK, 128), lambda b: (b, 0, 0)),
        out_shape=jax.ShapeDtypeStruct((B, NK, 128), F32),
    )(x, w6t, g6, b6, gm6, gm6t, w7at, w7bt, g7, b7, gm7, gm7t,
      w8t, g8, b8, gm8, gm8t)


def _groupmat(O):
    gsz = O // 32
    return (jnp.arange(O)[:, None] // gsz == jnp.arange(32)[None, :]).astype(F32)


def kernel(objpc, fpi, device, W1, g1, b1, W2, g2, b2, W3, g3, b3, W4, g4, b4,
           W5, g5, b5, W6, g6, b6, W7, g7, b7, W8, g8, b8):
    B, _, N = objpc.shape
    NK = fpi.shape[1]
    fpi = (fpi + jnp.asarray(device, fpi.dtype)).astype(jnp.int32)
    offs = jnp.arange(B, dtype=jnp.int32) * N

    xt = jnp.transpose(objpc, (0, 2, 1))
    xt128 = jnp.pad(xt, ((0, 0), (0, 0), (0, 122)))
    x3 = jnp.pad(xt[:, :, :3], ((0, 0), (0, 0), (0, 5)))
    x3t = jnp.transpose(x3, (0, 2, 1))

    G64 = _groupmat(64)
    G64t = G64.T

    w1t = jnp.pad(W1.T, ((0, 116), (0, 0)))
    xt128sh = jnp.roll(xt128, 6, axis=2)
    obj1_parts = []
    for b in range(B):
        x3b = lax.slice_in_dim(x3, b, b + 1, axis=0)
        x3tb = lax.slice_in_dim(x3t, b, b + 1, axis=0)
        xtb = lax.slice_in_dim(xt128, b, b + 1, axis=0)
        xtshb = lax.slice_in_dim(xt128sh, b, b + 1, axis=0)
        idx1b = _knn_call(x3b, x3tb, 512)[..., :KNB]
        idx1fb = jnp.transpose(idx1b, (0, 2, 1)).reshape(-1)
        nbr1b = _gather_rows(xtb.reshape(N, 128), idx1fb, 256)
        nbr1b = nbr1b.reshape(1, KNB * N, 128)
        y1b, st1b = _conv_stats_call(nbr1b, xtb, xtshb, w1t, nc=4)
        y2b, st2b = _norm_conv_stats_call(y1b, st1b, g1.reshape(1, -1),
                                          b1.reshape(1, -1), G64, G64t, W2.T,
                                          2.0 * N * KNB, nc=4)
        obj1_parts.append(
            _norm_maxk_call(y2b, st2b, g2.reshape(1, -1), b2.reshape(1, -1),
                            G64, G64t, 2.0 * N * KNB, M=N, nc=4))
    obj1 = jnp.concatenate(obj1_parts, axis=0)

    obj1p = jnp.pad(obj1, ((0, 0), (0, 0), (0, 64)))
    obj1_tab = obj1p.reshape(B * N, 128)
    obj1t = jnp.transpose(obj1, (0, 2, 1))
    fpig = (fpi + offs[:, None]).reshape(-1)
    o1g128 = _gather_rows(obj1_tab, fpig, 128).reshape(B, NK, 128)
    o1g = o1g128[:, :, :64]
    idx2 = _knn_call(o1g, obj1t, 512)[..., :KNB]
    idx2f = (jnp.transpose(idx2, (0, 2, 1)) + offs[:, None, None]).reshape(-1)
    nbr2 = _gather_rows(obj1_tab, idx2f, 512).reshape(B, KNB * NK, 128)
    y3, st3 = _conv_stats_call(nbr2, o1g128, jnp.roll(o1g128, 64, axis=2),
                               W3.T, nc=1)
    y4, st4 = _norm_conv_stats_call(y3, st3, g3.reshape(1, -1), b3.reshape(1, -1),
                                    G64, G64t, W4.T, 2.0 * NK * KNB, nc=1)
    obj2 = _norm_maxk_call(y4, st4, g4.reshape(1, -1), b4.reshape(1, -1),
                           G64, G64t, 2.0 * NK * KNB, M=NK, nc=1)

    offs2 = jnp.arange(B, dtype=jnp.int32) * NK
    obj2p = jnp.pad(obj2, ((0, 0), (0, 0), (0, 64)))
    obj2_tab = obj2p.reshape(B * NK, 128)
    obj2t = jnp.transpose(obj2, (0, 2, 1))
    idx3 = _knn_call(obj2, obj2t, 512)[..., :KNB]
    idx3f = (jnp.transpose(idx3, (0, 2, 1)) + offs2[:, None, None]).reshape(-1)
    nbr3 = _gather_rows(obj2_tab, idx3f, 512).reshape(B, KNB * NK, 128)
    y5, st5 = _conv_stats_call(nbr3, obj2p, jnp.roll(obj2p, 64, axis=2),
                               W5.T, nc=1)
    obj3 = _norm_maxk_call(y5, st5, g5.reshape(1, -1), b5.reshape(1, -1),
                           G64, G64t, 2.0 * NK * KNB, M=NK, nc=1)

    x192 = jnp.concatenate([o1g, obj2, obj3], axis=2)
    G6, G7, G8 = _groupmat(1024), _groupmat(512), _groupmat(128)
    out = _tail_call(x192, W6.T, g6.reshape(1, -1), b6.reshape(1, -1), G6, G6.T,
                     W7[:, :1024].T, W7[:, 1024:].T, g7.reshape(1, -1),
                     b7.reshape(1, -1), G7, G7.T,
                     W8.T, g8.reshape(1, -1), b8.reshape(1, -1), G8, G8.T)
    return jnp.transpose(out, (0, 2, 1))


# speedup vs baseline: 1.0797x; 1.0797x over previous
import functools

import jax
import jax.numpy as jnp
from jax import lax
from jax.experimental import pallas as pl
from jax.experimental.pallas import tpu as pltpu
from jax.experimental.pallas import tpu_sc as plsc

F32 = jnp.float32
KNB = 20
KPAD = 32
NEG = -3.0e38
EPS = 1e-5


def _gather_rows(table, idx, ch):
    V, D = table.shape
    (T,) = idx.shape
    info = plsc.get_sparse_core_info()
    nc_, ns_ = info.num_cores, info.num_subcores
    nw = nc_ * ns_
    per_w = T // nw
    assert T % nw == 0 and per_w % ch == 0 and ch % 8 == 0
    nch = per_w // ch
    mesh = plsc.VectorSubcoreMesh(core_axis_name="c", subcore_axis_name="s")

    @functools.partial(
        pl.kernel,
        mesh=mesh,
        out_type=jax.ShapeDtypeStruct((T, D), F32),
        scratch_types=[
            pltpu.VMEM((ch,), jnp.int32),
            pltpu.VMEM((ch, D), F32),
            pltpu.SemaphoreType.DMA,
        ],
    )
    def k(table_hbm, idx_hbm, out_hbm, idx_v, rows_v, sem):
        wid = lax.axis_index("s") * nc_ + lax.axis_index("c")
        base = wid * per_w
        for c in range(nch):
            off = base + c * ch
            pltpu.sync_copy(idx_hbm.at[pl.ds(off, ch)], idx_v)
            pltpu.async_copy(table_hbm.at[idx_v], rows_v, sem).wait()
            pltpu.sync_copy(rows_v, out_hbm.at[pl.ds(off, ch)])

    return k(table, idx)


def _knn_call(q, dbt, mb):
    B, M, C = q.shape
    N = dbt.shape[2]

    def body(q_ref, dbt_ref, o_ref):
        qb = q_ref[0]
        db = dbt_ref[0]
        pd = 2.0 * jnp.dot(qb, db, preferred_element_type=F32)
        pd = pd - jnp.sum(qb * qb, axis=1, keepdims=True)
        pd = pd - jnp.sum(db * db, axis=0, keepdims=True)
        ii = lax.broadcasted_iota(jnp.int32, (mb, N), 1)
        col = lax.broadcasted_iota(jnp.int32, (mb, KPAD), 1)
        acc = jnp.zeros((mb, KPAD), jnp.int32)
        for j in range(KNB):
            m = jnp.max(pd, axis=1, keepdims=True)
            am = jnp.min(jnp.where(pd == m, ii, N), axis=1, keepdims=True)
            acc = jnp.where(col == j, am, acc)
            pd = jnp.where(ii == am, NEG, pd)
        o_ref[0] = acc

    return pl.pallas_call(
        body,
        grid=(B, M // mb),
        in_specs=[
            pl.BlockSpec((1, mb, C), lambda b, m: (b, m, 0)),
            pl.BlockSpec((1, C, N), lambda b, m: (b, 0, 0)),
        ],
        out_specs=pl.BlockSpec((1, mb, KPAD), lambda b, m: (b, m, 0)),
        out_shape=jax.ShapeDtypeStruct((B, M, KPAD), jnp.int32),
    )(q, dbt)


def _lrelu(y):
    return jnp.where(y >= 0, y, 0.2 * y)


def _group_bcast(s, gm, gmt):
    return jnp.dot(jnp.dot(s, gm, preferred_element_type=F32,
                           precision=lax.Precision.HIGHEST), gmt,
                   preferred_element_type=F32, precision=lax.Precision.HIGHEST)


def _norm_act(y, s0, s1, gm, gmt, g, b, inv_cnt):
    mean = _group_bcast(s0, gm, gmt) * inv_cnt
    msq = _group_bcast(s1, gm, gmt) * inv_cnt
    var = msq - mean * mean
    return _lrelu((y - mean) / jnp.sqrt(var + EPS) * g + b)


def _conv_stats_call(nbr, ctr, ctrsh, wt, nc):
    B, S, C = nbr.shape
    M = ctr.shape[1]
    O = wt.shape[1]
    R = S // nc
    T = R // M

    def body(n_ref, c_ref, cs_ref, w_ref, y_ref, st_ref):
        ci = pl.program_id(1)
        ct = c_ref[0]
        csh = cs_ref[0]
        s0 = jnp.zeros((1, O), F32)
        s1 = jnp.zeros((1, O), F32)
        for t in range(T):
            x = n_ref[0, pl.ds(t * M, M), :]
            y = jnp.dot(x - ct + csh, w_ref[...], preferred_element_type=F32)
            y_ref[0, pl.ds(t * M, M), :] = y
            s0 = s0 + jnp.sum(y, axis=0, keepdims=True)
            s1 = s1 + jnp.sum(y * y, axis=0, keepdims=True)

        @pl.when(ci == 0)
        def _():
            st_ref[0, 0:1, :] = s0
            st_ref[0, 1:2, :] = s1

        @pl.when(ci != 0)
        def _():
            st_ref[0, 0:1, :] = st_ref[0, 0:1, :] + s0
            st_ref[0, 1:2, :] = st_ref[0, 1:2, :] + s1

    return pl.pallas_call(
        body,
        grid=(B, nc),
        in_specs=[
            pl.BlockSpec((1, R, C), lambda b, c: (b, c, 0)),
            pl.BlockSpec((1, M, C), lambda b, c: (b, 0, 0)),
            pl.BlockSpec((1, M, C), lambda b, c: (b, 0, 0)),
            pl.BlockSpec((C, O), lambda b, c: (0, 0)),
        ],
        out_specs=[
            pl.BlockSpec((1, R, O), lambda b, c: (b, c, 0)),
            pl.BlockSpec((1, 2, O), lambda b, c: (b, 0, 0)),
        ],
        out_shape=[
            jax.ShapeDtypeStruct((B, S, O), F32),
            jax.ShapeDtypeStruct((B, 2, O), F32),
        ],
    )(nbr, ctr, ctrsh, wt)


def _norm_conv_stats_call(y, st, g, b, gm, gmt, w2t, cnt, nc):
    B, S, O = y.shape
    O2 = w2t.shape[1]
    R = S // nc
    inv_cnt = 1.0 / cnt

    def body(y_ref, st_ref, g_ref, b_ref, gm_ref, gmt_ref, w_ref, y2_ref, st2_ref):
        ci = pl.program_id(1)
        a = _norm_act(y_ref[0], st_ref[0, 0:1, :], st_ref[0, 1:2, :],
                      gm_ref[...], gmt_ref[...], g_ref[...], b_ref[...], inv_cnt)
        y2 = jnp.dot(a, w_ref[...], preferred_element_type=F32)
        y2_ref[0] = y2
        s0 = jnp.sum(y2, axis=0, keepdims=True)
        s1 = jnp.sum(y2 * y2, axis=0, keepdims=True)

        @pl.when(ci == 0)
        def _():
            st2_ref[0, 0:1, :] = s0
            st2_ref[0, 1:2, :] = s1

        @pl.when(ci != 0)
        def _():
            st2_ref[0, 0:1, :] = st2_ref[0, 0:1, :] + s0
            st2_ref[0, 1:2, :] = st2_ref[0, 1:2, :] + s1

    return pl.pallas_call(
        body,
        grid=(B, nc),
        in_specs=[
            pl.BlockSpec((1, R, O), lambda b, c: (b, c, 0)),
            pl.BlockSpec((1, 2, O), lambda b, c: (b, 0, 0)),
            pl.BlockSpec((1, O), lambda b, c: (0, 0)),
            pl.BlockSpec((1, O), lambda b, c: (0, 0)),
            pl.BlockSpec((O, 32), lambda b, c: (0, 0)),
            pl.BlockSpec((32, O), lambda b, c: (0, 0)),
            pl.BlockSpec((O, O2), lambda b, c: (0, 0)),
        ],
        out_specs=[
            pl.BlockSpec((1, R, O2), lambda b, c: (b, c, 0)),
            pl.BlockSpec((1, 2, O2), lambda b, c: (b, 0, 0)),
        ],
        out_shape=[
            jax.ShapeDtypeStruct((B, S, O2), F32),
            jax.ShapeDtypeStruct((B, 2, O2), F32),
        ],
    )(y, st, g, b, gm, gmt, w2t)


def _norm_maxk_call(y, st, g, b, gm, gmt, cnt, M, nc):
    B, S, O = y.shape
    R = S // nc
    T = R // M
    inv_cnt = 1.0 / cnt

    def body(y_ref, st_ref, g_ref, b_ref, gm_ref, gmt_ref, o_ref):
        ci = pl.program_id(1)
        a = _norm_act(y_ref[0], st_ref[0, 0:1, :], st_ref[0, 1:2, :],
                      gm_ref[...], gmt_ref[...], g_ref[...], b_ref[...], inv_cnt)
        m = lax.slice(a, (0, 0), (M, O))
        for t in range(1, T):
            m = jnp.maximum(m, lax.slice(a, (t * M, 0), ((t + 1) * M, O)))

        @pl.when(ci == 0)
        def _():
            o_ref[0] = m

        @pl.when(ci != 0)
        def _():
            o_ref[0] = jnp.maximum(o_ref[0], m)

    return pl.pallas_call(
        body,
        grid=(B, nc),
        in_specs=[
            pl.BlockSpec((1, R, O), lambda b, c: (b, c, 0)),
            pl.BlockSpec((1, 2, O), lambda b, c: (b, 0, 0)),
            pl.BlockSpec((1, O), lambda b, c: (0, 0)),
            pl.BlockSpec((1, O), lambda b, c: (0, 0)),
            pl.BlockSpec((O, 32), lambda b, c: (0, 0)),
            pl.BlockSpec((32, O), lambda b, c: (0, 0)),
        ],
        out_specs=pl.BlockSpec((1, M, O), lambda b, c: (b, 0, 0)),
        out_shape=jax.ShapeDtypeStruct((B, M, O), F32),
    )(y, st, g, b, gm, gmt)


def _tail_call(x, w6t, g6, b6, gm6, gm6t, w7at, w7bt, g7, b7, gm7, gm7t,
               w8t, g8, b8, gm8, gm8t):
    B, NK, _ = x.shape

    def body(x_ref, w6_ref, g6_ref, b6_ref, gm6_ref, gm6t_ref,
             w7a_ref, w7b_ref, g7_ref, b7_ref, gm7_ref, gm7t_ref,
             w8_ref, g8_ref, b8_ref, gm8_ref, gm8t_ref, o_ref):
        xb = x_ref[0]
        y6 = jnp.dot(xb, w6_ref[...], preferred_element_type=F32)
        a6 = _norm_act(y6, jnp.sum(y6, axis=0, keepdims=True),
                       jnp.sum(y6 * y6, axis=0, keepdims=True),
                       gm6_ref[...], gm6t_ref[...], g6_ref[...], b6_ref[...],
                       1.0 / (32.0 * NK))
        gmax = jnp.max(a6, axis=0, keepdims=True)
        y7 = (jnp.dot(xb, w7b_ref[...], preferred_element_type=F32)
              + jnp.dot(gmax, w7a_ref[...], preferred_element_type=F32))
        a7 = _norm_act(y7, jnp.sum(y7, axis=0, keepdims=True),
                       jnp.sum(y7 * y7, axis=0, keepdims=True),
                       gm7_ref[...], gm7t_ref[...], g7_ref[...], b7_ref[...],
                       1.0 / (16.0 * NK))
        y8 = jnp.dot(a7, w8_ref[...], preferred_element_type=F32)
        a8 = _norm_act(y8, jnp.sum(y8, axis=0, keepdims=True),
                       jnp.sum(y8 * y8, axis=0, keepdims=True),
                       gm8_ref[...], gm8t_ref[...], g8_ref[...], b8_ref[...],
                       1.0 / (4.0 * NK))
        o_ref[0] = a8

    full = lambda shape: pl.BlockSpec(shape, lambda b: tuple(0 for _ in shape))
    return pl.pallas_call(
        body,
        grid=(B,),
        in_specs=[
            pl.BlockSpec((1, NK, 192), lambda b: (b, 0, 0)),
            full((192, 1024)), full((1, 1024)), full((1, 1024)),
            full((1024, 32)), full((32, 1024)),
            full((1024, 512)), full((192, 512)), full((1, 512)), full((1, 512)),
            full((512, 32)), full((32, 512)),
            full((512, 128)), full((1, 128)), full((1, 128)),
            full((128, 32)), full((32, 128)),
        ],
        out_specs=pl.BlockSpec((1, NK, 128), lambda b: (b, 0, 0)),
        out_shape=jax.ShapeDtypeStruct((B, NK, 128), F32),
    )(x, w6t, g6, b6, gm6, gm6t, w7at, w7bt, g7, b7, gm7, gm7t,
      w8t, g8, b8, gm8, gm8t)


def _groupmat(O):
    gsz = O // 32
    return (jnp.arange(O)[:, None] // gsz == jnp.arange(32)[None, :]).astype(F32)


def kernel(objpc, fpi, device, W1, g1, b1, W2, g2, b2, W3, g3, b3, W4, g4, b4,
           W5, g5, b5, W6, g6, b6, W7, g7, b7, W8, g8, b8):
    B, _, N = objpc.shape
    NK = fpi.shape[1]
    fpi = (fpi + jnp.asarray(device, fpi.dtype)).astype(jnp.int32)
    offs = jnp.arange(B, dtype=jnp.int32) * N

    xt = jnp.transpose(objpc, (0, 2, 1))
    xt128 = jnp.pad(xt, ((0, 0), (0, 0), (0, 122)))
    x3 = jnp.pad(xt[:, :, :3], ((0, 0), (0, 0), (0, 5)))
    x3t = jnp.transpose(x3, (0, 2, 1))

    G64 = _groupmat(64)
    G64t = G64.T

    idx1 = _knn_call(x3, x3t, 512)[..., :KNB]
    idx1f = (jnp.transpose(idx1, (0, 2, 1)) + offs[:, None, None]).reshape(-1)
    nbr1 = _gather_rows(xt128.reshape(B * N, 128), idx1f, 512)
    nbr1 = nbr1.reshape(B, KNB * N, 128)
    w1t = jnp.pad(W1.T, ((0, 116), (0, 0)))
    y1, st1 = _conv_stats_call(nbr1, xt128, jnp.roll(xt128, 6, axis=2),
                               w1t, nc=4)
    y2, st2 = _norm_conv_stats_call(y1, st1, g1.reshape(1, -1), b1.reshape(1, -1),
                                    G64, G64t, W2.T, 2.0 * N * KNB, nc=4)
    obj1 = _norm_maxk_call(y2, st2, g2.reshape(1, -1), b2.reshape(1, -1),
                           G64, G64t, 2.0 * N * KNB, M=N, nc=4)

    obj1p = jnp.pad(obj1, ((0, 0), (0, 0), (0, 64)))
    obj1_tab = obj1p.reshape(B * N, 128)
    obj1t = jnp.transpose(obj1, (0, 2, 1))
    fpig = (fpi + offs[:, None]).reshape(-1)
    o1g128 = _gather_rows(obj1_tab, fpig, 128).reshape(B, NK, 128)
    o1g = o1g128[:, :, :64]
    idx2 = _knn_call(o1g, obj1t, 512)[..., :KNB]
    idx2f = (jnp.transpose(idx2, (0, 2, 1)) + offs[:, None, None]).reshape(-1)
    nbr2 = _gather_rows(obj1_tab, idx2f, 512).reshape(B, KNB * NK, 128)
    y3, st3 = _conv_stats_call(nbr2, o1g128, jnp.roll(o1g128, 64, axis=2),
                               W3.T, nc=1)
    y4, st4 = _norm_conv_stats_call(y3, st3, g3.reshape(1, -1), b3.reshape(1, -1),
                                    G64, G64t, W4.T, 2.0 * NK * KNB, nc=1)
    obj2 = _norm_maxk_call(y4, st4, g4.reshape(1, -1), b4.reshape(1, -1),
                           G64, G64t, 2.0 * NK * KNB, M=NK, nc=1)

    offs2 = jnp.arange(B, dtype=jnp.int32) * NK
    obj2p = jnp.pad(obj2, ((0, 0), (0, 0), (0, 64)))
    obj2_tab = obj2p.reshape(B * NK, 128)
    obj2t = jnp.transpose(obj2, (0, 2, 1))
    idx3 = _knn_call(obj2, obj2t, 512)[..., :KNB]
    idx3f = (jnp.transpose(idx3, (0, 2, 1)) + offs2[:, None, None]).reshape(-1)
    nbr3 = _gather_rows(obj2_tab, idx3f, 512).reshape(B, KNB * NK, 128)
    y5, st5 = _conv_stats_call(nbr3, obj2p, jnp.roll(obj2p, 64, axis=2),
                               W5.T, nc=1)
    obj3 = _norm_maxk_call(y5, st5, g5.reshape(1, -1), b5.reshape(1, -1),
                           G64, G64t, 2.0 * NK * KNB, M=NK, nc=1)

    x192 = jnp.concatenate([o1g, obj2, obj3], axis=2)
    G6, G7, G8 = _groupmat(1024), _groupmat(512), _groupmat(128)
    out = _tail_call(x192, W6.T, g6.reshape(1, -1), b6.reshape(1, -1), G6, G6.T,
                     W7[:, :1024].T, W7[:, 1024:].T, g7.reshape(1, -1),
                     b7.reshape(1, -1), G7, G7.T,
                     W8.T, g8.reshape(1, -1), b8.reshape(1, -1), G8, G8.T)
    return jnp.transpose(out, (0, 2, 1))
